# baseline (speedup 1.0000x reference)
import functools

import jax
import jax.numpy as jnp
from jax.experimental import pallas as pl

_BM = 512


def _act(y, act):
    if act == "lrelu":
        return jnp.where(y > 0, y, 0.01 * y)
    return y


def _mm1_kfn(act, x_ref, w_ref, b_ref, o_ref):
    y = jnp.dot(x_ref[...], w_ref[...], preferred_element_type=jnp.float32)
    o_ref[...] = _act(y + b_ref[...], act)


def _mm1(x, w, b, act=None):
    m, k = x.shape
    n = w.shape[1]
    return pl.pallas_call(
        functools.partial(_mm1_kfn, act),
        grid=(pl.cdiv(m, _BM),),
        in_specs=[
            pl.BlockSpec((_BM, k), lambda i: (i, 0)),
            pl.BlockSpec((k, n), lambda i: (0, 0)),
            pl.BlockSpec((1, n), lambda i: (0, 0)),
        ],
        out_specs=pl.BlockSpec((_BM, n), lambda i: (i, 0)),
        out_shape=jax.ShapeDtypeStruct((m, n), jnp.float32),
    )(x, w, b.reshape(1, -1))


def _mm2_kfn(act, x1_ref, x2_ref, w1_ref, w2_ref, b_ref, o_ref):
    y = jnp.dot(x1_ref[...], w1_ref[...], preferred_element_type=jnp.float32)
    y += jnp.dot(x2_ref[...], w2_ref[...], preferred_element_type=jnp.float32)
    o_ref[...] = _act(y + b_ref[...], act)


def _mm2(x1, x2, w, b, act=None):
    m, k1 = x1.shape
    k2 = x2.shape[1]
    n = w.shape[1]
    w1, w2 = w[:k1], w[k1:]
    return pl.pallas_call(
        functools.partial(_mm2_kfn, act),
        grid=(pl.cdiv(m, _BM),),
        in_specs=[
            pl.BlockSpec((_BM, k1), lambda i: (i, 0)),
            pl.BlockSpec((_BM, k2), lambda i: (i, 0)),
            pl.BlockSpec((k1, n), lambda i: (0, 0)),
            pl.BlockSpec((k2, n), lambda i: (0, 0)),
            pl.BlockSpec((1, n), lambda i: (0, 0)),
        ],
        out_specs=pl.BlockSpec((_BM, n), lambda i: (i, 0)),
        out_shape=jax.ShapeDtypeStruct((m, n), jnp.float32),
    )(x1, x2, w1, w2, b.reshape(1, -1))


def _gru_kfn(h_dim, x_ref, h_ref, wx_ref, wh_ref, bx_ref, bh_ref, o_ref):
    gx = jnp.dot(x_ref[...], wx_ref[...], preferred_element_type=jnp.float32)
    gx += bx_ref[...]
    gh = jnp.dot(h_ref[...], wh_ref[...], preferred_element_type=jnp.float32)
    gh += bh_ref[...]
    hd = h_dim
    r = jax.nn.sigmoid(gx[:, :hd] + gh[:, :hd])
    z = jax.nn.sigmoid(gx[:, hd:2 * hd] + gh[:, hd:2 * hd])
    n = jnp.tanh(gx[:, 2 * hd:] + r * gh[:, 2 * hd:])
    o_ref[...] = (1.0 - z) * n + z * h_ref[...]


def _gru_cell(x, h, wx, wh, bx, bh):
    m, hd = h.shape
    return pl.pallas_call(
        functools.partial(_gru_kfn, hd),
        grid=(pl.cdiv(m, _BM),),
        in_specs=[
            pl.BlockSpec((_BM, hd), lambda i: (i, 0)),
            pl.BlockSpec((_BM, hd), lambda i: (i, 0)),
            pl.BlockSpec((hd, 3 * hd), lambda i: (0, 0)),
            pl.BlockSpec((hd, 3 * hd), lambda i: (0, 0)),
            pl.BlockSpec((1, 3 * hd), lambda i: (0, 0)),
            pl.BlockSpec((1, 3 * hd), lambda i: (0, 0)),
        ],
        out_specs=pl.BlockSpec((_BM, hd), lambda i: (i, 0)),
        out_shape=jax.ShapeDtypeStruct((m, hd), jnp.float32),
    )(x, h, wx, wh, bx.reshape(1, -1), bh.reshape(1, -1))


def _seg_softmax(logits, seg, num):
    mx = jax.ops.segment_max(logits, seg, num_segments=num)
    mx = jnp.where(jnp.isfinite(mx), mx, 0.0)
    e = jnp.exp(logits - mx[seg])
    d = jax.ops.segment_sum(e, seg, num_segments=num)
    return e / (d[seg] + 1e-12)


def _bnorm(y, g, b):
    m = y.mean(0)
    v = y.var(0)
    return (y - m) / jnp.sqrt(v + 1e-5) * g + b


def kernel(origin_node, origin_edge, edge_index, node_batch, params):
    lrelu = jax.nn.leaky_relu
    src, dst = edge_index[0], edge_index[1]
    n_nodes = origin_node.shape[0]
    n_graphs = 256

    h0 = lrelu(_bnorm(_mm1(origin_node, params["node_w"], params["node_b"]),
                      params["node_g"], params["node_beta"]))
    e0 = lrelu(_bnorm(_mm1(origin_edge, params["edge_w"], params["edge_b"]),
                      params["edge_g"], params["edge_beta"]))

    outs = []
    for p in params["heads"]:
        h = h0
        hs = h[src]
        m = _mm2(hs, e0, p["W1"], p["b1"], act="lrelu")
        lg = _mm2(h[dst], m, p["W2"], p["b2"], act="lrelu")
        a = _seg_softmax(lg[:, 0], dst, n_nodes)[:, None]
        mv = _mm1(m, p["W3"], p["b3"])
        c = jax.nn.elu(jax.ops.segment_sum(a * mv, dst, num_segments=n_nodes))
        h = _gru_cell(c, h, p["g1Wx"], p["g1Wh"], p["g1bx"], p["g1bh"])
        m2 = _mm1(h[src], p["W1b"], p["b1b"], act="lrelu")
        lg2 = _mm2(h[dst], m2, p["W2b"], p["b2b"], act="lrelu")
        a2 = _seg_softmax(lg2[:, 0], dst, n_nodes)[:, None]
        m2v = _mm1(m2, p["W3b"], p["b3b"])
        c2 = jax.nn.elu(jax.ops.segment_sum(a2 * m2v, dst, num_segments=n_nodes))
        h = _gru_cell(c2, h, p["g2Wx"], p["g2Wh"], p["g2bx"], p["g2bh"])
        s = jax.ops.segment_sum(h, node_batch, num_segments=n_graphs)
        hw = _mm1(h, p["Wb"], p["bb"])
        for _ in range(2):
            lgm = _mm2(s[node_batch], h, p["Wa"], p["ba"], act="lrelu")
            am = _seg_softmax(lgm[:, 0], node_batch, n_graphs)[:, None]
            cm = jax.nn.elu(jax.ops.segment_sum(am * hw, node_batch,
                                                num_segments=n_graphs))
            s = _gru_cell(cm, s, p["gmWx"], p["gmWh"], p["gmbx"], p["gmbh"])
        outs.append(s)

    cat = jnp.concatenate(outs, axis=1)
    y = _mm1(cat, params["att_w"], params["att_b"])
    return jax.nn.relu(_bnorm(y, params["att_g"], params["att_beta"]))


# speedup vs baseline: 1.1127x; 1.1127x over previous
import functools

import jax
import jax.numpy as jnp
from jax.experimental import pallas as pl

_BM = 512


def _act(y, act):
    if act == "lrelu":
        return jnp.where(y > 0, y, 0.01 * y)
    return y


def _mm1_kfn(act, x_ref, w_ref, b_ref, o_ref):
    y = jnp.dot(x_ref[...], w_ref[...], preferred_element_type=jnp.float32)
    o_ref[...] = _act(y + b_ref[...], act)


def _mm1(x, w, b, act=None):
    m, k = x.shape
    n = w.shape[1]
    return pl.pallas_call(
        functools.partial(_mm1_kfn, act),
        grid=(pl.cdiv(m, _BM),),
        in_specs=[
            pl.BlockSpec((_BM, k), lambda i: (i, 0)),
            pl.BlockSpec((k, n), lambda i: (0, 0)),
            pl.BlockSpec((1, n), lambda i: (0, 0)),
        ],
        out_specs=pl.BlockSpec((_BM, n), lambda i: (i, 0)),
        out_shape=jax.ShapeDtypeStruct((m, n), jnp.float32),
    )(x, w, b.reshape(1, -1))


def _mm2_kfn(act, x1_ref, x2_ref, w1_ref, w2_ref, b_ref, o_ref):
    y = jnp.dot(x1_ref[...], w1_ref[...], preferred_element_type=jnp.float32)
    y += jnp.dot(x2_ref[...], w2_ref[...], preferred_element_type=jnp.float32)
    o_ref[...] = _act(y + b_ref[...], act)


def _mm2(x1, x2, w, b, act=None):
    m, k1 = x1.shape
    k2 = x2.shape[1]
    n = w.shape[1]
    w1, w2 = w[:k1], w[k1:]
    return pl.pallas_call(
        functools.partial(_mm2_kfn, act),
        grid=(pl.cdiv(m, _BM),),
        in_specs=[
            pl.BlockSpec((_BM, k1), lambda i: (i, 0)),
            pl.BlockSpec((_BM, k2), lambda i: (i, 0)),
            pl.BlockSpec((k1, n), lambda i: (0, 0)),
            pl.BlockSpec((k2, n), lambda i: (0, 0)),
            pl.BlockSpec((1, n), lambda i: (0, 0)),
        ],
        out_specs=pl.BlockSpec((_BM, n), lambda i: (i, 0)),
        out_shape=jax.ShapeDtypeStruct((m, n), jnp.float32),
    )(x1, x2, w1, w2, b.reshape(1, -1))


def _edge1_kfn(hs_ref, hd_ref, e0_ref, w1a_ref, w1b_ref, b1_ref,
               w2a_ref, w2b_ref, b2_ref, w3_ref, b3_ref, lg_ref, mv_ref):
    m = jnp.dot(hs_ref[...], w1a_ref[...], preferred_element_type=jnp.float32)
    m += jnp.dot(e0_ref[...], w1b_ref[...], preferred_element_type=jnp.float32)
    m = _act(m + b1_ref[...], "lrelu")
    lg = jnp.dot(hd_ref[...], w2a_ref[...], preferred_element_type=jnp.float32)
    lg += jnp.dot(m, w2b_ref[...], preferred_element_type=jnp.float32)
    lg_ref[...] = _act(lg + b2_ref[...], "lrelu")
    mv = jnp.dot(m, w3_ref[...], preferred_element_type=jnp.float32)
    mv_ref[...] = mv + b3_ref[...]


def _edge1(hs, hd, e0, w1, b1, w2, b2, w3, b3):
    m, k = hs.shape
    row = lambda i: (i, 0)
    full = lambda i: (0, 0)
    return pl.pallas_call(
        _edge1_kfn,
        grid=(pl.cdiv(m, _BM),),
        in_specs=[
            pl.BlockSpec((_BM, k), row),
            pl.BlockSpec((_BM, k), row),
            pl.BlockSpec((_BM, k), row),
            pl.BlockSpec((k, k), full),
            pl.BlockSpec((k, k), full),
            pl.BlockSpec((1, k), full),
            pl.BlockSpec((k, 1), full),
            pl.BlockSpec((k, 1), full),
            pl.BlockSpec((1, 1), full),
            pl.BlockSpec((k, k), full),
            pl.BlockSpec((1, k), full),
        ],
        out_specs=[pl.BlockSpec((_BM, 1), row), pl.BlockSpec((_BM, k), row)],
        out_shape=[jax.ShapeDtypeStruct((m, 1), jnp.float32),
                   jax.ShapeDtypeStruct((m, k), jnp.float32)],
    )(hs, hd, e0, w1[:k], w1[k:], b1.reshape(1, -1),
      w2[:k], w2[k:], b2.reshape(1, -1), w3, b3.reshape(1, -1))


def _edge2_kfn(hs_ref, hd_ref, w1_ref, b1_ref,
               w2a_ref, w2b_ref, b2_ref, w3_ref, b3_ref, lg_ref, mv_ref):
    m = jnp.dot(hs_ref[...], w1_ref[...], preferred_element_type=jnp.float32)
    m = _act(m + b1_ref[...], "lrelu")
    lg = jnp.dot(hd_ref[...], w2a_ref[...], preferred_element_type=jnp.float32)
    lg += jnp.dot(m, w2b_ref[...], preferred_element_type=jnp.float32)
    lg_ref[...] = _act(lg + b2_ref[...], "lrelu")
    mv = jnp.dot(m, w3_ref[...], preferred_element_type=jnp.float32)
    mv_ref[...] = mv + b3_ref[...]


def _edge2(hs, hd, w1, b1, w2, b2, w3, b3):
    m, k = hs.shape
    row = lambda i: (i, 0)
    full = lambda i: (0, 0)
    return pl.pallas_call(
        _edge2_kfn,
        grid=(pl.cdiv(m, _BM),),
        in_specs=[
            pl.BlockSpec((_BM, k), row),
            pl.BlockSpec((_BM, k), row),
            pl.BlockSpec((k, k), full),
            pl.BlockSpec((1, k), full),
            pl.BlockSpec((k, 1), full),
            pl.BlockSpec((k, 1), full),
            pl.BlockSpec((1, 1), full),
            pl.BlockSpec((k, k), full),
            pl.BlockSpec((1, k), full),
        ],
        out_specs=[pl.BlockSpec((_BM, 1), row), pl.BlockSpec((_BM, k), row)],
        out_shape=[jax.ShapeDtypeStruct((m, 1), jnp.float32),
                   jax.ShapeDtypeStruct((m, k), jnp.float32)],
    )(hs, hd, w1, b1.reshape(1, -1),
      w2[:k], w2[k:], b2.reshape(1, -1), w3, b3.reshape(1, -1))


def _gru_kfn(h_dim, x_ref, h_ref, wx_ref, wh_ref, bx_ref, bh_ref, o_ref):
    gx = jnp.dot(x_ref[...], wx_ref[...], preferred_element_type=jnp.float32)
    gx += bx_ref[...]
    gh = jnp.dot(h_ref[...], wh_ref[...], preferred_element_type=jnp.float32)
    gh += bh_ref[...]
    hd = h_dim
    r = jax.nn.sigmoid(gx[:, :hd] + gh[:, :hd])
    z = jax.nn.sigmoid(gx[:, hd:2 * hd] + gh[:, hd:2 * hd])
    n = jnp.tanh(gx[:, 2 * hd:] + r * gh[:, 2 * hd:])
    o_ref[...] = (1.0 - z) * n + z * h_ref[...]


def _gru_cell(x, h, wx, wh, bx, bh):
    m, hd = h.shape
    return pl.pallas_call(
        functools.partial(_gru_kfn, hd),
        grid=(pl.cdiv(m, _BM),),
        in_specs=[
            pl.BlockSpec((_BM, hd), lambda i: (i, 0)),
            pl.BlockSpec((_BM, hd), lambda i: (i, 0)),
            pl.BlockSpec((hd, 3 * hd), lambda i: (0, 0)),
            pl.BlockSpec((hd, 3 * hd), lambda i: (0, 0)),
            pl.BlockSpec((1, 3 * hd), lambda i: (0, 0)),
            pl.BlockSpec((1, 3 * hd), lambda i: (0, 0)),
        ],
        out_specs=pl.BlockSpec((_BM, hd), lambda i: (i, 0)),
        out_shape=jax.ShapeDtypeStruct((m, hd), jnp.float32),
    )(x, h, wx, wh, bx.reshape(1, -1), bh.reshape(1, -1))


def _seg_softmax(logits, seg, num):
    mx = jax.ops.segment_max(logits, seg, num_segments=num)
    mx = jnp.where(jnp.isfinite(mx), mx, 0.0)
    e = jnp.exp(logits - mx[seg])
    d = jax.ops.segment_sum(e, seg, num_segments=num)
    return e / (d[seg] + 1e-12)


def _bnorm(y, g, b):
    m = y.mean(0)
    v = y.var(0)
    return (y - m) / jnp.sqrt(v + 1e-5) * g + b


def kernel(origin_node, origin_edge, edge_index, node_batch, params):
    lrelu = jax.nn.leaky_relu
    src, dst = edge_index[0], edge_index[1]
    n_nodes = origin_node.shape[0]
    n_graphs = 256

    h0 = lrelu(_bnorm(_mm1(origin_node, params["node_w"], params["node_b"]),
                      params["node_g"], params["node_beta"]))
    e0 = lrelu(_bnorm(_mm1(origin_edge, params["edge_w"], params["edge_b"]),
                      params["edge_g"], params["edge_beta"]))

    outs = []
    for p in params["heads"]:
        h = h0
        lg, mv = _edge1(h[src], h[dst], e0, p["W1"], p["b1"],
                        p["W2"], p["b2"], p["W3"], p["b3"])
        a = _seg_softmax(lg[:, 0], dst, n_nodes)[:, None]
        c = jax.nn.elu(jax.ops.segment_sum(a * mv, dst, num_segments=n_nodes))
        h = _gru_cell(c, h, p["g1Wx"], p["g1Wh"], p["g1bx"], p["g1bh"])
        lg2, m2v = _edge2(h[src], h[dst], p["W1b"], p["b1b"],
                          p["W2b"], p["b2b"], p["W3b"], p["b3b"])
        a2 = _seg_softmax(lg2[:, 0], dst, n_nodes)[:, None]
        c2 = jax.nn.elu(jax.ops.segment_sum(a2 * m2v, dst, num_segments=n_nodes))
        h = _gru_cell(c2, h, p["g2Wx"], p["g2Wh"], p["g2bx"], p["g2bh"])
        s = jax.ops.segment_sum(h, node_batch, num_segments=n_graphs)
        hw = _mm1(h, p["Wb"], p["bb"])
        for _ in range(2):
            lgm = _mm2(s[node_batch], h, p["Wa"], p["ba"], act="lrelu")
            am = _seg_softmax(lgm[:, 0], node_batch, n_graphs)[:, None]
            cm = jax.nn.elu(jax.ops.segment_sum(am * hw, node_batch,
                                                num_segments=n_graphs))
            s = _gru_cell(cm, s, p["gmWx"], p["gmWh"], p["gmbx"], p["gmbh"])
        outs.append(s)

    cat = jnp.concatenate(outs, axis=1)
    y = _mm1(cat, params["att_w"], params["att_b"])
    return jax.nn.relu(_bnorm(y, params["att_g"], params["att_beta"]))


# speedup vs baseline: 1.1505x; 1.0340x over previous
import functools

import jax
import jax.numpy as jnp
from jax.experimental import pallas as pl

_BM = 1024


def _act(y, act):
    if act == "lrelu":
        return jnp.where(y > 0, y, 0.01 * y)
    return y


def _mm1_kfn(act, x_ref, w_ref, b_ref, o_ref):
    y = jnp.dot(x_ref[...], w_ref[...], preferred_element_type=jnp.float32)
    o_ref[...] = _act(y + b_ref[...], act)


def _mm1(x, w, b, act=None):
    m, k = x.shape
    n = w.shape[1]
    return pl.pallas_call(
        functools.partial(_mm1_kfn, act),
        grid=(pl.cdiv(m, _BM),),
        in_specs=[
            pl.BlockSpec((_BM, k), lambda i: (i, 0)),
            pl.BlockSpec((k, n), lambda i: (0, 0)),
            pl.BlockSpec((1, n), lambda i: (0, 0)),
        ],
        out_specs=pl.BlockSpec((_BM, n), lambda i: (i, 0)),
        out_shape=jax.ShapeDtypeStruct((m, n), jnp.float32),
    )(x, w, b.reshape(1, -1))


def _mm2_kfn(act, x1_ref, x2_ref, w1_ref, w2_ref, b_ref, o_ref):
    y = jnp.dot(x1_ref[...], w1_ref[...], preferred_element_type=jnp.float32)
    y += jnp.dot(x2_ref[...], w2_ref[...], preferred_element_type=jnp.float32)
    o_ref[...] = _act(y + b_ref[...], act)


def _mm2(x1, x2, w, b, act=None):
    m, k1 = x1.shape
    k2 = x2.shape[1]
    n = w.shape[1]
    w1, w2 = w[:k1], w[k1:]
    return pl.pallas_call(
        functools.partial(_mm2_kfn, act),
        grid=(pl.cdiv(m, _BM),),
        in_specs=[
            pl.BlockSpec((_BM, k1), lambda i: (i, 0)),
            pl.BlockSpec((_BM, k2), lambda i: (i, 0)),
            pl.BlockSpec((k1, n), lambda i: (0, 0)),
            pl.BlockSpec((k2, n), lambda i: (0, 0)),
            pl.BlockSpec((1, n), lambda i: (0, 0)),
        ],
        out_specs=pl.BlockSpec((_BM, n), lambda i: (i, 0)),
        out_shape=jax.ShapeDtypeStruct((m, n), jnp.float32),
    )(x1, x2, w1, w2, b.reshape(1, -1))


def _edge1_kfn(hs_ref, hd_ref, e0_ref, w1a_ref, w1b_ref, b1_ref,
               w2a_ref, w2b_ref, b2_ref, w3_ref, b3_ref, lg_ref, mv_ref):
    m = jnp.dot(hs_ref[...], w1a_ref[...], preferred_element_type=jnp.float32)
    m += jnp.dot(e0_ref[...], w1b_ref[...], preferred_element_type=jnp.float32)
    m = _act(m + b1_ref[...], "lrelu")
    lg = jnp.dot(hd_ref[...], w2a_ref[...], preferred_element_type=jnp.float32)
    lg += jnp.dot(m, w2b_ref[...], preferred_element_type=jnp.float32)
    lg_ref[...] = _act(lg + b2_ref[...], "lrelu")
    mv = jnp.dot(m, w3_ref[...], preferred_element_type=jnp.float32)
    mv_ref[...] = mv + b3_ref[...]


def _edge1(hs, hd, e0, w1, b1, w2, b2, w3, b3):
    m, k = hs.shape
    row = lambda i: (i, 0)
    full = lambda i: (0, 0)
    return pl.pallas_call(
        _edge1_kfn,
        grid=(pl.cdiv(m, _BM),),
        in_specs=[
            pl.BlockSpec((_BM, k), row),
            pl.BlockSpec((_BM, k), row),
            pl.BlockSpec((_BM, k), row),
            pl.BlockSpec((k, k), full),
            pl.BlockSpec((k, k), full),
            pl.BlockSpec((1, k), full),
            pl.BlockSpec((k, 1), full),
            pl.BlockSpec((k, 1), full),
            pl.BlockSpec((1, 1), full),
            pl.BlockSpec((k, k), full),
            pl.BlockSpec((1, k), full),
        ],
        out_specs=[pl.BlockSpec((_BM, 1), row), pl.BlockSpec((_BM, k), row)],
        out_shape=[jax.ShapeDtypeStruct((m, 1), jnp.float32),
                   jax.ShapeDtypeStruct((m, k), jnp.float32)],
    )(hs, hd, e0, w1[:k], w1[k:], b1.reshape(1, -1),
      w2[:k], w2[k:], b2.reshape(1, -1), w3, b3.reshape(1, -1))


def _edge2_kfn(hs_ref, hd_ref, w1_ref, b1_ref,
               w2a_ref, w2b_ref, b2_ref, w3_ref, b3_ref, lg_ref, mv_ref):
    m = jnp.dot(hs_ref[...], w1_ref[...], preferred_element_type=jnp.float32)
    m = _act(m + b1_ref[...], "lrelu")
    lg = jnp.dot(hd_ref[...], w2a_ref[...], preferred_element_type=jnp.float32)
    lg += jnp.dot(m, w2b_ref[...], preferred_element_type=jnp.float32)
    lg_ref[...] = _act(lg + b2_ref[...], "lrelu")
    mv = jnp.dot(m, w3_ref[...], preferred_element_type=jnp.float32)
    mv_ref[...] = mv + b3_ref[...]


def _edge2(hs, hd, w1, b1, w2, b2, w3, b3):
    m, k = hs.shape
    row = lambda i: (i, 0)
    full = lambda i: (0, 0)
    return pl.pallas_call(
        _edge2_kfn,
        grid=(pl.cdiv(m, _BM),),
        in_specs=[
            pl.BlockSpec((_BM, k), row),
            pl.BlockSpec((_BM, k), row),
            pl.BlockSpec((k, k), full),
            pl.BlockSpec((1, k), full),
            pl.BlockSpec((k, 1), full),
            pl.BlockSpec((k, 1), full),
            pl.BlockSpec((1, 1), full),
            pl.BlockSpec((k, k), full),
            pl.BlockSpec((1, k), full),
        ],
        out_specs=[pl.BlockSpec((_BM, 1), row), pl.BlockSpec((_BM, k), row)],
        out_shape=[jax.ShapeDtypeStruct((m, 1), jnp.float32),
                   jax.ShapeDtypeStruct((m, k), jnp.float32)],
    )(hs, hd, w1, b1.reshape(1, -1),
      w2[:k], w2[k:], b2.reshape(1, -1), w3, b3.reshape(1, -1))


def _gru_kfn(h_dim, x_ref, h_ref, wx_ref, wh_ref, bx_ref, bh_ref, o_ref):
    gx = jnp.dot(x_ref[...], wx_ref[...], preferred_element_type=jnp.float32)
    gx += bx_ref[...]
    gh = jnp.dot(h_ref[...], wh_ref[...], preferred_element_type=jnp.float32)
    gh += bh_ref[...]
    hd = h_dim
    r = jax.nn.sigmoid(gx[:, :hd] + gh[:, :hd])
    z = jax.nn.sigmoid(gx[:, hd:2 * hd] + gh[:, hd:2 * hd])
    n = jnp.tanh(gx[:, 2 * hd:] + r * gh[:, 2 * hd:])
    o_ref[...] = (1.0 - z) * n + z * h_ref[...]


def _gru_cell(x, h, wx, wh, bx, bh):
    m, hd = h.shape
    return pl.pallas_call(
        functools.partial(_gru_kfn, hd),
        grid=(pl.cdiv(m, _BM),),
        in_specs=[
            pl.BlockSpec((_BM, hd), lambda i: (i, 0)),
            pl.BlockSpec((_BM, hd), lambda i: (i, 0)),
            pl.BlockSpec((hd, 3 * hd), lambda i: (0, 0)),
            pl.BlockSpec((hd, 3 * hd), lambda i: (0, 0)),
            pl.BlockSpec((1, 3 * hd), lambda i: (0, 0)),
            pl.BlockSpec((1, 3 * hd), lambda i: (0, 0)),
        ],
        out_specs=pl.BlockSpec((_BM, hd), lambda i: (i, 0)),
        out_shape=jax.ShapeDtypeStruct((m, hd), jnp.float32),
    )(x, h, wx, wh, bx.reshape(1, -1), bh.reshape(1, -1))


def _seg_softmax(logits, seg, num):
    mx = jax.ops.segment_max(logits, seg, num_segments=num)
    mx = jnp.where(jnp.isfinite(mx), mx, 0.0)
    e = jnp.exp(logits - mx[seg])
    d = jax.ops.segment_sum(e, seg, num_segments=num)
    return e / (d[seg] + 1e-12)


def _bnorm(y, g, b):
    m = y.mean(0)
    v = y.var(0)
    return (y - m) / jnp.sqrt(v + 1e-5) * g + b


def kernel(origin_node, origin_edge, edge_index, node_batch, params):
    lrelu = jax.nn.leaky_relu
    src, dst = edge_index[0], edge_index[1]
    n_nodes = origin_node.shape[0]
    n_graphs = 256

    h0 = lrelu(_bnorm(_mm1(origin_node, params["node_w"], params["node_b"]),
                      params["node_g"], params["node_beta"]))
    e0 = lrelu(_bnorm(_mm1(origin_edge, params["edge_w"], params["edge_b"]),
                      params["edge_g"], params["edge_beta"]))

    outs = []
    for p in params["heads"]:
        h = h0
        lg, mv = _edge1(h[src], h[dst], e0, p["W1"], p["b1"],
                        p["W2"], p["b2"], p["W3"], p["b3"])
        a = _seg_softmax(lg[:, 0], dst, n_nodes)[:, None]
        c = jax.nn.elu(jax.ops.segment_sum(a * mv, dst, num_segments=n_nodes))
        h = _gru_cell(c, h, p["g1Wx"], p["g1Wh"], p["g1bx"], p["g1bh"])
        lg2, m2v = _edge2(h[src], h[dst], p["W1b"], p["b1b"],
                          p["W2b"], p["b2b"], p["W3b"], p["b3b"])
        a2 = _seg_softmax(lg2[:, 0], dst, n_nodes)[:, None]
        c2 = jax.nn.elu(jax.ops.segment_sum(a2 * m2v, dst, num_segments=n_nodes))
        h = _gru_cell(c2, h, p["g2Wx"], p["g2Wh"], p["g2bx"], p["g2bh"])
        s = jax.ops.segment_sum(h, node_batch, num_segments=n_graphs)
        hw = _mm1(h, p["Wb"], p["bb"])
        for _ in range(2):
            lgm = _mm2(s[node_batch], h, p["Wa"], p["ba"], act="lrelu")
            am = _seg_softmax(lgm[:, 0], node_batch, n_graphs)[:, None]
            cm = jax.nn.elu(jax.ops.segment_sum(am * hw, node_batch,
                                                num_segments=n_graphs))
            s = _gru_cell(cm, s, p["gmWx"], p["gmWh"], p["gmbx"], p["gmbh"])
        outs.append(s)

    cat = jnp.concatenate(outs, axis=1)
    y = _mm1(cat, params["att_w"], params["att_b"])
    return jax.nn.relu(_bnorm(y, params["att_g"], params["att_beta"]))


# speedup vs baseline: 1.1707x; 1.0176x over previous
import functools

import jax
import jax.numpy as jnp
from jax.experimental import pallas as pl

_BM = 2048


def _act(y, act):
    if act == "lrelu":
        return jnp.where(y > 0, y, 0.01 * y)
    return y


def _mm1_kfn(act, x_ref, w_ref, b_ref, o_ref):
    y = jnp.dot(x_ref[...], w_ref[...], preferred_element_type=jnp.float32)
    o_ref[...] = _act(y + b_ref[...], act)


def _mm1(x, w, b, act=None):
    m, k = x.shape
    n = w.shape[1]
    return pl.pallas_call(
        functools.partial(_mm1_kfn, act),
        grid=(pl.cdiv(m, _BM),),
        in_specs=[
            pl.BlockSpec((_BM, k), lambda i: (i, 0)),
            pl.BlockSpec((k, n), lambda i: (0, 0)),
            pl.BlockSpec((1, n), lambda i: (0, 0)),
        ],
        out_specs=pl.BlockSpec((_BM, n), lambda i: (i, 0)),
        out_shape=jax.ShapeDtypeStruct((m, n), jnp.float32),
    )(x, w, b.reshape(1, -1))


def _mm2_kfn(act, x1_ref, x2_ref, w1_ref, w2_ref, b_ref, o_ref):
    y = jnp.dot(x1_ref[...], w1_ref[...], preferred_element_type=jnp.float32)
    y += jnp.dot(x2_ref[...], w2_ref[...], preferred_element_type=jnp.float32)
    o_ref[...] = _act(y + b_ref[...], act)


def _mm2(x1, x2, w, b, act=None):
    m, k1 = x1.shape
    k2 = x2.shape[1]
    n = w.shape[1]
    w1, w2 = w[:k1], w[k1:]
    return pl.pallas_call(
        functools.partial(_mm2_kfn, act),
        grid=(pl.cdiv(m, _BM),),
        in_specs=[
            pl.BlockSpec((_BM, k1), lambda i: (i, 0)),
            pl.BlockSpec((_BM, k2), lambda i: (i, 0)),
            pl.BlockSpec((k1, n), lambda i: (0, 0)),
            pl.BlockSpec((k2, n), lambda i: (0, 0)),
            pl.BlockSpec((1, n), lambda i: (0, 0)),
        ],
        out_specs=pl.BlockSpec((_BM, n), lambda i: (i, 0)),
        out_shape=jax.ShapeDtypeStruct((m, n), jnp.float32),
    )(x1, x2, w1, w2, b.reshape(1, -1))


def _edge1_kfn(hs_ref, hd_ref, e0_ref, w1a_ref, w1b_ref, b1_ref,
               w2a_ref, w2b_ref, b2_ref, w3_ref, b3_ref, lg_ref, mv_ref):
    m = jnp.dot(hs_ref[...], w1a_ref[...], preferred_element_type=jnp.float32)
    m += jnp.dot(e0_ref[...], w1b_ref[...], preferred_element_type=jnp.float32)
    m = _act(m + b1_ref[...], "lrelu")
    lg = jnp.dot(hd_ref[...], w2a_ref[...], preferred_element_type=jnp.float32)
    lg += jnp.dot(m, w2b_ref[...], preferred_element_type=jnp.float32)
    lg_ref[...] = _act(lg + b2_ref[...], "lrelu")
    mv = jnp.dot(m, w3_ref[...], preferred_element_type=jnp.float32)
    mv_ref[...] = mv + b3_ref[...]


def _edge1(hs, hd, e0, w1, b1, w2, b2, w3, b3):
    m, k = hs.shape
    row = lambda i: (i, 0)
    full = lambda i: (0, 0)
    return pl.pallas_call(
        _edge1_kfn,
        grid=(pl.cdiv(m, _BM),),
        in_specs=[
            pl.BlockSpec((_BM, k), row),
            pl.BlockSpec((_BM, k), row),
            pl.BlockSpec((_BM, k), row),
            pl.BlockSpec((k, k), full),
            pl.BlockSpec((k, k), full),
            pl.BlockSpec((1, k), full),
            pl.BlockSpec((k, 1), full),
            pl.BlockSpec((k, 1), full),
            pl.BlockSpec((1, 1), full),
            pl.BlockSpec((k, k), full),
            pl.BlockSpec((1, k), full),
        ],
        out_specs=[pl.BlockSpec((_BM, 1), row), pl.BlockSpec((_BM, k), row)],
        out_shape=[jax.ShapeDtypeStruct((m, 1), jnp.float32),
                   jax.ShapeDtypeStruct((m, k), jnp.float32)],
    )(hs, hd, e0, w1[:k], w1[k:], b1.reshape(1, -1),
      w2[:k], w2[k:], b2.reshape(1, -1), w3, b3.reshape(1, -1))


def _edge2_kfn(hs_ref, hd_ref, w1_ref, b1_ref,
               w2a_ref, w2b_ref, b2_ref, w3_ref, b3_ref, lg_ref, mv_ref):
    m = jnp.dot(hs_ref[...], w1_ref[...], preferred_element_type=jnp.float32)
    m = _act(m + b1_ref[...], "lrelu")
    lg = jnp.dot(hd_ref[...], w2a_ref[...], preferred_element_type=jnp.float32)
    lg += jnp.dot(m, w2b_ref[...], preferred_element_type=jnp.float32)
    lg_ref[...] = _act(lg + b2_ref[...], "lrelu")
    mv = jnp.dot(m, w3_ref[...], preferred_element_type=jnp.float32)
    mv_ref[...] = mv + b3_ref[...]


def _edge2(hs, hd, w1, b1, w2, b2, w3, b3):
    m, k = hs.shape
    row = lambda i: (i, 0)
    full = lambda i: (0, 0)
    return pl.pallas_call(
        _edge2_kfn,
        grid=(pl.cdiv(m, _BM),),
        in_specs=[
            pl.BlockSpec((_BM, k), row),
            pl.BlockSpec((_BM, k), row),
            pl.BlockSpec((k, k), full),
            pl.BlockSpec((1, k), full),
            pl.BlockSpec((k, 1), full),
            pl.BlockSpec((k, 1), full),
            pl.BlockSpec((1, 1), full),
            pl.BlockSpec((k, k), full),
            pl.BlockSpec((1, k), full),
        ],
        out_specs=[pl.BlockSpec((_BM, 1), row), pl.BlockSpec((_BM, k), row)],
        out_shape=[jax.ShapeDtypeStruct((m, 1), jnp.float32),
                   jax.ShapeDtypeStruct((m, k), jnp.float32)],
    )(hs, hd, w1, b1.reshape(1, -1),
      w2[:k], w2[k:], b2.reshape(1, -1), w3, b3.reshape(1, -1))


def _gru_kfn(h_dim, x_ref, h_ref, wx_ref, wh_ref, bx_ref, bh_ref, o_ref):
    gx = jnp.dot(x_ref[...], wx_ref[...], preferred_element_type=jnp.float32)
    gx += bx_ref[...]
    gh = jnp.dot(h_ref[...], wh_ref[...], preferred_element_type=jnp.float32)
    gh += bh_ref[...]
    hd = h_dim
    r = jax.nn.sigmoid(gx[:, :hd] + gh[:, :hd])
    z = jax.nn.sigmoid(gx[:, hd:2 * hd] + gh[:, hd:2 * hd])
    n = jnp.tanh(gx[:, 2 * hd:] + r * gh[:, 2 * hd:])
    o_ref[...] = (1.0 - z) * n + z * h_ref[...]


def _gru_cell(x, h, wx, wh, bx, bh):
    m, hd = h.shape
    return pl.pallas_call(
        functools.partial(_gru_kfn, hd),
        grid=(pl.cdiv(m, _BM),),
        in_specs=[
            pl.BlockSpec((_BM, hd), lambda i: (i, 0)),
            pl.BlockSpec((_BM, hd), lambda i: (i, 0)),
            pl.BlockSpec((hd, 3 * hd), lambda i: (0, 0)),
            pl.BlockSpec((hd, 3 * hd), lambda i: (0, 0)),
            pl.BlockSpec((1, 3 * hd), lambda i: (0, 0)),
            pl.BlockSpec((1, 3 * hd), lambda i: (0, 0)),
        ],
        out_specs=pl.BlockSpec((_BM, hd), lambda i: (i, 0)),
        out_shape=jax.ShapeDtypeStruct((m, hd), jnp.float32),
    )(x, h, wx, wh, bx.reshape(1, -1), bh.reshape(1, -1))


def _seg_softmax(logits, seg, num):
    mx = jax.ops.segment_max(logits, seg, num_segments=num)
    mx = jnp.where(jnp.isfinite(mx), mx, 0.0)
    e = jnp.exp(logits - mx[seg])
    d = jax.ops.segment_sum(e, seg, num_segments=num)
    return e / (d[seg] + 1e-12)


def _bnorm(y, g, b):
    m = y.mean(0)
    v = y.var(0)
    return (y - m) / jnp.sqrt(v + 1e-5) * g + b


def kernel(origin_node, origin_edge, edge_index, node_batch, params):
    lrelu = jax.nn.leaky_relu
    src, dst = edge_index[0], edge_index[1]
    n_nodes = origin_node.shape[0]
    n_graphs = 256

    h0 = lrelu(_bnorm(_mm1(origin_node, params["node_w"], params["node_b"]),
                      params["node_g"], params["node_beta"]))
    e0 = lrelu(_bnorm(_mm1(origin_edge, params["edge_w"], params["edge_b"]),
                      params["edge_g"], params["edge_beta"]))

    outs = []
    for p in params["heads"]:
        h = h0
        lg, mv = _edge1(h[src], h[dst], e0, p["W1"], p["b1"],
                        p["W2"], p["b2"], p["W3"], p["b3"])
        a = _seg_softmax(lg[:, 0], dst, n_nodes)[:, None]
        c = jax.nn.elu(jax.ops.segment_sum(a * mv, dst, num_segments=n_nodes))
        h = _gru_cell(c, h, p["g1Wx"], p["g1Wh"], p["g1bx"], p["g1bh"])
        lg2, m2v = _edge2(h[src], h[dst], p["W1b"], p["b1b"],
                          p["W2b"], p["b2b"], p["W3b"], p["b3b"])
        a2 = _seg_softmax(lg2[:, 0], dst, n_nodes)[:, None]
        c2 = jax.nn.elu(jax.ops.segment_sum(a2 * m2v, dst, num_segments=n_nodes))
        h = _gru_cell(c2, h, p["g2Wx"], p["g2Wh"], p["g2bx"], p["g2bh"])
        s = jax.ops.segment_sum(h, node_batch, num_segments=n_graphs)
        hw = _mm1(h, p["Wb"], p["bb"])
        for _ in range(2):
            lgm = _mm2(s[node_batch], h, p["Wa"], p["ba"], act="lrelu")
            am = _seg_softmax(lgm[:, 0], node_batch, n_graphs)[:, None]
            cm = jax.nn.elu(jax.ops.segment_sum(am * hw, node_batch,
                                                num_segments=n_graphs))
            s = _gru_cell(cm, s, p["gmWx"], p["gmWh"], p["gmbx"], p["gmbh"])
        outs.append(s)

    cat = jnp.concatenate(outs, axis=1)
    y = _mm1(cat, params["att_w"], params["att_b"])
    return jax.nn.relu(_bnorm(y, params["att_g"], params["att_beta"]))
